# baseline (speedup 1.0000x reference)
import jax
import jax.numpy as jnp
from jax.experimental import pallas as pl
from jax.experimental.pallas import tpu as pltpu
from jax.experimental.pallas import tpu_sc as plsc

_NUM_SC = 2
_NUM_SUBCORES = 16


def _sc_gather(table, ids_flat, window=128):
    n = ids_flat.shape[0]
    hid = table.shape[1]
    n_sub = _NUM_SC * _NUM_SUBCORES
    per_sub = n // n_sub
    n_windows = per_sub // window
    mesh = plsc.VectorSubcoreMesh(core_axis_name="c", subcore_axis_name="s")

    @pl.kernel(
        out_type=jax.ShapeDtypeStruct((n, hid), table.dtype),
        mesh=mesh,
        scratch_types=[
            pltpu.VMEM((1, per_sub), jnp.int32),
            pltpu.VMEM((window, hid), table.dtype),
        ],
    )
    def gather_kernel(x_hbm, i_hbm, o_hbm, idx_buf, row_buf):
        c = jax.lax.axis_index("c")
        s = jax.lax.axis_index("s")
        sub = c * _NUM_SUBCORES + s
        base = sub * per_sub
        pltpu.sync_copy(i_hbm.at[0, pl.ds(base, per_sub)], idx_buf.at[0])
        for w in range(n_windows):
            pltpu.sync_copy(
                x_hbm.at[idx_buf.at[0, pl.ds(w * window, window)]], row_buf
            )
            pltpu.sync_copy(row_buf, o_hbm.at[pl.ds(base + w * window, window)])

    return gather_kernel(table, ids_flat.reshape(1, n))


def _ln_body(x_ref, p_ref, g_ref, bta_ref, o_ref):
    x = x_ref[0] + p_ref[...]
    mean = jnp.mean(x, axis=-1, keepdims=True)
    xc = x - mean
    var = jnp.mean(xc * xc, axis=-1, keepdims=True)
    o_ref[0] = (xc * jax.lax.rsqrt(var + 1e-5)) * g_ref[...] + bta_ref[...]


def _tc_add_ln_chunk(prev, gchunk, pos, gamma2d, beta2d, b_total, b_start, blk):
    cb, s, h = gchunk.shape

    specs = [
        pl.BlockSpec((1, blk, h), lambda i, j: (j, i, 0)),
        pl.BlockSpec((blk, h), lambda i, j: (i, 0)),
        pl.BlockSpec((1, h), lambda i, j: (0, 0)),
        pl.BlockSpec((1, h), lambda i, j: (0, 0)),
    ]
    out_spec = pl.BlockSpec(
        (1, blk, h), lambda i, j, b_start=b_start: (b_start + j, i, 0)
    )
    out_shape = jax.ShapeDtypeStruct((b_total, s, h), gchunk.dtype)

    if prev is None:
        return pl.pallas_call(
            _ln_body,
            grid=(s // blk, cb),
            in_specs=specs,
            out_specs=out_spec,
            out_shape=out_shape,
        )(gchunk, pos, gamma2d, beta2d)

    def body(prev_ref, x_ref, p_ref, g_ref, bta_ref, o_ref):
        del prev_ref
        _ln_body(x_ref, p_ref, g_ref, bta_ref, o_ref)

    return pl.pallas_call(
        body,
        grid=(s // blk, cb),
        in_specs=[pl.BlockSpec(memory_space=pl.ANY)] + specs,
        out_specs=out_spec,
        out_shape=out_shape,
        input_output_aliases={0: 0},
    )(prev, gchunk, pos, gamma2d, beta2d)


def kernel(input_ids, word_embeddings, position_embeddings, ln_gamma, ln_beta):
    b, s = input_ids.shape
    hid = word_embeddings.shape[1]
    ids_flat = input_ids.reshape(-1).astype(jnp.int32)
    pos = position_embeddings[:s]
    gamma2d = ln_gamma.reshape(1, hid)
    beta2d = ln_beta.reshape(1, hid)

    chunk_b = 2
    n_chunks = b // chunk_b
    gathered = [
        _sc_gather(
            word_embeddings,
            jax.lax.dynamic_slice_in_dim(ids_flat, k * chunk_b * s, chunk_b * s),
        ).reshape(chunk_b, s, hid)
        for k in range(n_chunks)
    ]
    out = None
    for k in range(n_chunks):
        out = _tc_add_ln_chunk(
            out, gathered[k], pos, gamma2d, beta2d, b, k * chunk_b, blk=512
        )
    return out
